# baseline (speedup 1.0000x reference)
import jax
import jax.numpy as jnp
import numpy as np
from jax.experimental import pallas as pl
from jax.experimental.pallas import tpu as pltpu

BT = 256


def _net_kernel(xp_ref, t1_ref, t2_ref, fw1_ref, fb1_ref, fw2_ref, fb2_ref,
                out_ref):
    xb = xp_ref[...]
    x2d = jnp.transpose(xb, (1, 0, 2)).reshape(40 * BT, 32)
    xcat = jnp.concatenate(
        [x2d[0:32 * BT], x2d[BT:33 * BT], x2d[2 * BT:34 * BT]], axis=1)
    parts = []
    for g in range(6):
        h1 = jnp.dot(xcat, t1_ref[g], preferred_element_type=jnp.float32)
        x1 = jnp.maximum(h1.astype(jnp.bfloat16), 0)
        acc = None
        for d in range(3):
            xs = x1[d * BT:(d + 24) * BT]
            p = jnp.dot(xs, t2_ref[d], preferred_element_type=jnp.float32)
            acc = p if acc is None else acc + p
        h2 = jnp.maximum(acc, 0.0)
        m = jnp.max(h2.reshape(12, 2, BT, 256), axis=1)
        p1 = jnp.maximum(m[..., 0:64], m[..., 64:128])
        p2 = jnp.maximum(m[..., 128:192], m[..., 192:256])
        parts.append(jnp.concatenate([p1, p2], axis=2).astype(jnp.bfloat16))
    flat = jnp.concatenate(parts, axis=2)
    flat = flat.reshape(12 * BT, 768)

    facc = None
    for t in range(12):
        ft = jnp.dot(flat[t * BT:(t + 1) * BT], fw1_ref[t],
                     preferred_element_type=jnp.float32)
        facc = ft if facc is None else facc + ft
    f1 = jnp.maximum(facc + fb1_ref[...], 0.0).astype(jnp.bfloat16)
    f2 = jnp.dot(f1, fw2_ref[...], preferred_element_type=jnp.float32)
    out_ref[...] = (f2 + fb2_ref[...])[:, :10]


_SEL1 = np.zeros((32, 3, 6, 8), np.float32)
_SEL2 = np.zeros((8, 3, 4), np.float32)
_B1 = np.zeros((3, 32), np.float32)
_ONE6 = np.zeros((6, 8), np.float32)
_ONESLOT = np.zeros((3, 32, 6, 8, 32), np.float32)
_S6 = np.zeros((8, 32), np.float32)
for _g in range(6):
    for _s in range(6):
        for _e in range(3):
            _SEL1[4 * _g + _s + _e, _e, _g, _s] = 1.0
    _ONE6[_g, :6] = 1.0
    _ONESLOT[0, 31, _g, 6, 0] = 1.0
for _s in range(8):
    for _e in range(3):
        _q = _s - _e
        if 0 <= _q <= 3 and _s <= 5:
            _SEL2[_s, _e, _q] = 1.0
_B1[0, 31] = 1.0
_S6[6, 0] = 1.0
_D0 = np.array([1.0, 0.0, 0.0], np.float32)
_Q1 = np.ones(4, np.float32)


def _prep(conv1_w, conv1_b, conv2_w, conv2_b, fc1_w, fc1_b, fc2_w, fc2_b):
    w1r = conv1_w[:, 0, :, :]
    t1 = (jnp.einsum('cde,pegs->dpgsc', w1r, _SEL1)
          + jnp.einsum('dp,gs,c->dpgsc', _B1, _ONE6, conv1_b)
          + _ONESLOT).reshape(96, 6, 256).transpose(1, 0, 2)
    t2 = (jnp.einsum('oide,seq->dsiqo', conv2_w, _SEL2)
          + jnp.einsum('d,si,q,o->dsiqo', _D0, _S6, _Q1, conv2_b))
    t2 = t2.reshape(3, 256, 256)
    fw1 = fc1_w.reshape(128, 64, 12, 12).transpose(2, 3, 1, 0)
    fw1 = fw1.reshape(12, 768, 128)
    fw2 = jnp.zeros((128, 128), jnp.float32).at[:, :10].set(fc2_w.T)
    fb2 = jnp.zeros((1, 128), jnp.float32).at[0, :10].set(fc2_b)
    return (t1.astype(jnp.bfloat16), t2.astype(jnp.bfloat16),
            fw1.astype(jnp.bfloat16), fc1_b.reshape(1, 128),
            fw2.astype(jnp.bfloat16), fb2)


def _call(xp, args, interpret=False):
    b = xp.shape[0]
    grid = b // BT
    const = lambda *shape: pl.BlockSpec(shape, lambda i: (0,) * len(shape))
    return pl.pallas_call(
        _net_kernel,
        grid=(grid,),
        in_specs=[
            pl.BlockSpec((BT, 40, 32), lambda i: (i, 0, 0)),
            const(6, 96, 256), const(3, 256, 256), const(12, 768, 128),
            const(1, 128), const(128, 128), const(1, 128),
        ],
        out_specs=pl.BlockSpec((BT, 10), lambda i: (i, 0)),
        out_shape=jax.ShapeDtypeStruct((b, 10), jnp.float32),
        interpret=interpret,
    )(xp, *args)


def kernel(x, conv1_w, conv1_b, conv2_w, conv2_b, fc1_w, fc1_b, fc2_w, fc2_b):
    args = _prep(conv1_w, conv1_b, conv2_w, conv2_b,
                 fc1_w, fc1_b, fc2_w, fc2_b)
    xp = jnp.pad(x.reshape(x.shape[0], 28, 28), ((0, 0), (0, 12), (0, 4)))
    xp = xp.at[:, :, 31].set(1.0).astype(jnp.bfloat16)
    return _call(xp, args)


# speedup vs baseline: 1.2352x; 1.2352x over previous
import jax
import jax.numpy as jnp
import numpy as np
from jax.experimental import pallas as pl
from jax.experimental.pallas import tpu as pltpu

BT = 256


def _net_kernel(xp_ref, t1_ref, t2_ref, fw1_ref, fb1_ref, fw2_ref, fb2_ref,
                out_ref):
    x2d = xp_ref[...].reshape(40 * BT, 32)
    xcat = jnp.concatenate(
        [x2d[0:32 * BT], x2d[BT:33 * BT], x2d[2 * BT:34 * BT]], axis=1)
    parts = []
    for g in range(6):
        h1 = jnp.dot(xcat, t1_ref[g], preferred_element_type=jnp.float32)
        x1 = jnp.maximum(h1.astype(jnp.bfloat16), 0)
        xs3 = jnp.concatenate(
            [x1[0:24 * BT], x1[BT:25 * BT], x1[2 * BT:26 * BT]], axis=1)
        acc = jnp.dot(xs3, t2_ref[...], preferred_element_type=jnp.float32)
        h2 = jnp.maximum(acc, 0.0)
        m = jnp.max(h2.reshape(12, 2, BT, 256), axis=1)
        p1 = jnp.maximum(m[..., 0:64], m[..., 64:128])
        p2 = jnp.maximum(m[..., 128:192], m[..., 192:256])
        parts.append(jnp.concatenate([p1, p2], axis=2).astype(jnp.bfloat16))
    flat = jnp.concatenate(parts, axis=2)
    flat = flat.reshape(12 * BT, 768)

    facc = None
    for t in range(12):
        ft = jnp.dot(flat[t * BT:(t + 1) * BT], fw1_ref[t],
                     preferred_element_type=jnp.float32)
        facc = ft if facc is None else facc + ft
    f1 = jnp.maximum(facc + fb1_ref[...], 0.0).astype(jnp.bfloat16)
    f2 = jnp.dot(f1, fw2_ref[...], preferred_element_type=jnp.float32)
    out_ref[...] = (f2 + fb2_ref[...])[:, :10]


_SEL1 = np.zeros((32, 3, 6, 8), np.float32)
_SEL2 = np.zeros((8, 3, 4), np.float32)
_B1 = np.zeros((3, 32), np.float32)
_ONE6 = np.zeros((6, 8), np.float32)
_ONESLOT = np.zeros((3, 32, 6, 8, 32), np.float32)
_S6 = np.zeros((8, 32), np.float32)
for _g in range(6):
    for _s in range(6):
        for _e in range(3):
            _SEL1[4 * _g + _s + _e, _e, _g, _s] = 1.0
    _ONE6[_g, :6] = 1.0
    _ONESLOT[0, 31, _g, 6, 0] = 1.0
for _s in range(8):
    for _e in range(3):
        _q = _s - _e
        if 0 <= _q <= 3 and _s <= 5:
            _SEL2[_s, _e, _q] = 1.0
_B1[0, 31] = 1.0
_S6[6, 0] = 1.0
_D0 = np.array([1.0, 0.0, 0.0], np.float32)
_Q1 = np.ones(4, np.float32)


def _prep(conv1_w, conv1_b, conv2_w, conv2_b, fc1_w, fc1_b, fc2_w, fc2_b):
    w1r = conv1_w[:, 0, :, :]
    t1 = (jnp.einsum('cde,pegs->dpgsc', w1r, _SEL1)
          + jnp.einsum('dp,gs,c->dpgsc', _B1, _ONE6, conv1_b)
          + _ONESLOT).reshape(96, 6, 256).transpose(1, 0, 2)
    t2 = (jnp.einsum('oide,seq->dsiqo', conv2_w, _SEL2)
          + jnp.einsum('d,si,q,o->dsiqo', _D0, _S6, _Q1, conv2_b))
    t2 = t2.reshape(768, 256)
    fw1 = fc1_w.reshape(128, 64, 12, 12).transpose(2, 3, 1, 0)
    fw1 = fw1.reshape(12, 768, 128)
    fw2 = jnp.zeros((128, 128), jnp.float32).at[:, :10].set(fc2_w.T)
    fb2 = jnp.zeros((1, 128), jnp.float32).at[0, :10].set(fc2_b)
    return (t1.astype(jnp.bfloat16), t2.astype(jnp.bfloat16),
            fw1.astype(jnp.bfloat16), fc1_b.reshape(1, 128),
            fw2.astype(jnp.bfloat16), fb2)


def _call(xp, args, interpret=False):
    b = xp.shape[1]
    grid = b // BT
    const = lambda *shape: pl.BlockSpec(shape, lambda i: (0,) * len(shape))
    return pl.pallas_call(
        _net_kernel,
        grid=(grid,),
        in_specs=[
            pl.BlockSpec((40, BT, 32), lambda i: (0, i, 0)),
            const(6, 96, 256), const(768, 256), const(12, 768, 128),
            const(1, 128), const(128, 128), const(1, 128),
        ],
        out_specs=pl.BlockSpec((BT, 10), lambda i: (i, 0)),
        out_shape=jax.ShapeDtypeStruct((b, 10), jnp.float32),
        interpret=interpret,
    )(xp, *args)


def kernel(x, conv1_w, conv1_b, conv2_w, conv2_b, fc1_w, fc1_b, fc2_w, fc2_b):
    args = _prep(conv1_w, conv1_b, conv2_w, conv2_b,
                 fc1_w, fc1_b, fc2_w, fc2_b)
    xp = jnp.pad(x.reshape(x.shape[0], 28, 28), ((0, 0), (0, 12), (0, 4)))
    xp = xp.at[:, :, 31].set(1.0)
    xp = xp.transpose(1, 0, 2).astype(jnp.bfloat16)
    return _call(xp, args)


# speedup vs baseline: 1.3181x; 1.0671x over previous
import jax
import jax.numpy as jnp
import numpy as np
from jax.experimental import pallas as pl
from jax.experimental.pallas import tpu as pltpu

BT = 256


def _net_kernel(xp_ref, t1_ref, t2_ref, b2_ref, fw1_ref, fb1_ref, fw2_ref,
                fb2_ref, out_ref):
    x2d = xp_ref[...].reshape(40 * BT, 32)
    xcat = jnp.concatenate(
        [x2d[0:32 * BT], x2d[BT:33 * BT], x2d[2 * BT:34 * BT]], axis=1)
    h1 = jnp.dot(xcat, t1_ref[...], preferred_element_type=jnp.float32)
    x1 = jnp.maximum(h1.astype(jnp.bfloat16), 0)
    parts = []
    for g in range(6):
        sl = slice(128 * g, 128 * g + 192)
        xs3 = jnp.concatenate(
            [x1[0:24 * BT, sl], x1[BT:25 * BT, sl], x1[2 * BT:26 * BT, sl]],
            axis=1)
        acc = jnp.dot(xs3, t2_ref[...], preferred_element_type=jnp.float32)
        h2 = jnp.maximum(acc + b2_ref[...], 0.0)
        m = jnp.max(h2.reshape(12, 2, BT, 256), axis=1)
        p1 = jnp.maximum(m[..., 0:64], m[..., 64:128])
        p2 = jnp.maximum(m[..., 128:192], m[..., 192:256])
        parts.append(jnp.concatenate([p1, p2], axis=2).astype(jnp.bfloat16))
    flat = jnp.concatenate(parts, axis=2)
    flat = flat.reshape(12 * BT, 768)

    facc = None
    for t in range(12):
        ft = jnp.dot(flat[t * BT:(t + 1) * BT], fw1_ref[t],
                     preferred_element_type=jnp.float32)
        facc = ft if facc is None else facc + ft
    f1 = jnp.maximum(facc + fb1_ref[...], 0.0).astype(jnp.bfloat16)
    f2 = jnp.dot(f1, fw2_ref[...], preferred_element_type=jnp.float32)
    out_ref[...] = (f2 + fb2_ref[...])[:, :10]


_SEL1 = np.zeros((32, 3, 26), np.float32)
_SEL2 = np.zeros((6, 3, 4), np.float32)
_B1 = np.zeros((3, 32), np.float32)
for _j in range(26):
    for _e in range(3):
        _SEL1[_j + _e, _e, _j] = 1.0
for _s in range(6):
    for _e in range(3):
        _q = _s - _e
        if 0 <= _q <= 3:
            _SEL2[_s, _e, _q] = 1.0
_B1[0, 31] = 1.0
_ONE26 = np.ones(26, np.float32)


def _prep(conv1_w, conv1_b, conv2_w, conv2_b, fc1_w, fc1_b, fc2_w, fc2_b):
    w1r = conv1_w[:, 0, :, :]
    t1 = (jnp.einsum('cde,pej->dpjc', w1r, _SEL1)
          + jnp.einsum('dp,j,c->dpjc', _B1, _ONE26, conv1_b)).reshape(96, 832)
    t2 = jnp.einsum('oide,seq->dsiqo', conv2_w, _SEL2).reshape(576, 256)
    b2 = jnp.tile(conv2_b, 4).reshape(1, 256)
    fw1 = fc1_w.reshape(128, 64, 12, 12).transpose(2, 3, 1, 0)
    fw1 = fw1.reshape(12, 768, 128)
    fw2 = jnp.zeros((128, 128), jnp.float32).at[:, :10].set(fc2_w.T)
    fb2 = jnp.zeros((1, 128), jnp.float32).at[0, :10].set(fc2_b)
    return (t1.astype(jnp.bfloat16), t2.astype(jnp.bfloat16), b2,
            fw1.astype(jnp.bfloat16), fc1_b.reshape(1, 128),
            fw2.astype(jnp.bfloat16), fb2)


def _call(xp, args, interpret=False):
    b = xp.shape[1]
    grid = b // BT
    const = lambda *shape: pl.BlockSpec(shape, lambda i: (0,) * len(shape))
    return pl.pallas_call(
        _net_kernel,
        grid=(grid,),
        in_specs=[
            pl.BlockSpec((40, BT, 32), lambda i: (0, i, 0)),
            const(96, 832), const(576, 256), const(1, 256),
            const(12, 768, 128),
            const(1, 128), const(128, 128), const(1, 128),
        ],
        out_specs=pl.BlockSpec((BT, 10), lambda i: (i, 0)),
        out_shape=jax.ShapeDtypeStruct((b, 10), jnp.float32),
        interpret=interpret,
    )(xp, *args)


def kernel(x, conv1_w, conv1_b, conv2_w, conv2_b, fc1_w, fc1_b, fc2_w, fc2_b):
    args = _prep(conv1_w, conv1_b, conv2_w, conv2_b,
                 fc1_w, fc1_b, fc2_w, fc2_b)
    xp = jnp.pad(x.reshape(x.shape[0], 28, 28), ((0, 0), (0, 12), (0, 4)))
    xp = xp.at[:, :, 31].set(1.0)
    xp = xp.transpose(1, 0, 2).astype(jnp.bfloat16)
    return _call(xp, args)


# speedup vs baseline: 1.3189x; 1.0006x over previous
import jax
import jax.numpy as jnp
import numpy as np
from jax.experimental import pallas as pl

BT = 256


def _net_kernel(xp_ref, t1_ref, t2_ref, b2_ref, fw1_ref, fb1_ref, fw2_ref,
                fb2_ref, out_ref):
    x2d = xp_ref[...].reshape(40 * BT, 32)
    xcat = jnp.concatenate(
        [x2d[0:32 * BT], x2d[BT:33 * BT], x2d[2 * BT:34 * BT]], axis=1)
    h1 = jnp.dot(xcat, t1_ref[...], preferred_element_type=jnp.float32)
    x1 = jnp.maximum(h1.astype(jnp.bfloat16), 0)
    parts = []
    for g in range(6):
        sl = slice(128 * g, 128 * g + 192)
        xs3 = jnp.concatenate(
            [x1[0:24 * BT, sl], x1[BT:25 * BT, sl], x1[2 * BT:26 * BT, sl]],
            axis=1)
        acc = jnp.dot(xs3, t2_ref[...], preferred_element_type=jnp.float32)
        h2 = jnp.maximum(acc + b2_ref[...], 0.0)
        m = jnp.max(h2.reshape(12, 2, BT, 256), axis=1)
        p1 = jnp.maximum(m[..., 0:64], m[..., 64:128])
        p2 = jnp.maximum(m[..., 128:192], m[..., 192:256])
        parts.append(jnp.concatenate([p1, p2], axis=2).astype(jnp.bfloat16))
    flat = jnp.concatenate(parts, axis=2)
    flat = flat.reshape(12 * BT, 768)

    facc = None
    for t in range(12):
        ft = jnp.dot(flat[t * BT:(t + 1) * BT], fw1_ref[t],
                     preferred_element_type=jnp.float32)
        facc = ft if facc is None else facc + ft
    f1 = jnp.maximum(facc + fb1_ref[...], 0.0).astype(jnp.bfloat16)
    f2 = jnp.dot(f1, fw2_ref[...], preferred_element_type=jnp.float32)
    out_ref[...] = (f2 + fb2_ref[...])[:, :10]


_SEL1 = np.zeros((32, 3, 26), np.float32)
_SEL2 = np.zeros((6, 3, 4), np.float32)
_B1 = np.zeros((3, 32), np.float32)
for _j in range(26):
    for _e in range(3):
        _SEL1[_j + _e, _e, _j] = 1.0
for _s in range(6):
    for _e in range(3):
        _q = _s - _e
        if 0 <= _q <= 3:
            _SEL2[_s, _e, _q] = 1.0
_B1[0, 31] = 1.0
_ONE26 = np.ones(26, np.float32)


def _prep(conv1_w, conv1_b, conv2_w, conv2_b, fc1_w, fc1_b, fc2_w, fc2_b):
    w1r = conv1_w[:, 0, :, :]
    t1 = (jnp.einsum('cde,pej->dpjc', w1r, _SEL1)
          + jnp.einsum('dp,j,c->dpjc', _B1, _ONE26, conv1_b)).reshape(96, 832)
    t2 = jnp.einsum('oide,seq->dsiqo', conv2_w, _SEL2).reshape(576, 256)
    b2 = jnp.tile(conv2_b, 4).reshape(1, 256)
    fw1 = fc1_w.reshape(128, 64, 12, 12).transpose(2, 3, 1, 0)
    fw1 = fw1.reshape(12, 768, 128)
    fw2 = jnp.zeros((128, 128), jnp.float32).at[:, :10].set(fc2_w.T)
    fb2 = jnp.zeros((1, 128), jnp.float32).at[0, :10].set(fc2_b)
    return (t1.astype(jnp.bfloat16), t2.astype(jnp.bfloat16), b2,
            fw1.astype(jnp.bfloat16), fc1_b.reshape(1, 128),
            fw2.astype(jnp.bfloat16), fb2)


def _call(xp, args):
    b = xp.shape[1]
    grid = b // BT
    const = lambda *shape: pl.BlockSpec(shape, lambda i: (0,) * len(shape))
    return pl.pallas_call(
        _net_kernel,
        grid=(grid,),
        in_specs=[
            pl.BlockSpec((40, BT, 32), lambda i: (0, i, 0)),
            const(96, 832), const(576, 256), const(1, 256),
            const(12, 768, 128),
            const(1, 128), const(128, 128), const(1, 128),
        ],
        out_specs=pl.BlockSpec((BT, 10), lambda i: (i, 0)),
        out_shape=jax.ShapeDtypeStruct((b, 10), jnp.float32),
    )(xp, *args)


def kernel(x, conv1_w, conv1_b, conv2_w, conv2_b, fc1_w, fc1_b, fc2_w, fc2_b):
    args = _prep(conv1_w, conv1_b, conv2_w, conv2_b,
                 fc1_w, fc1_b, fc2_w, fc2_b)
    xp = jnp.pad(x.reshape(x.shape[0], 28, 28), ((0, 0), (0, 12), (0, 4)))
    xp = xp.at[:, :, 31].set(1.0)
    xp = xp.transpose(1, 0, 2).astype(jnp.bfloat16)
    return _call(xp, args)
